# baseline (speedup 1.0000x reference)
import functools

import jax
import jax.numpy as jnp
from jax import lax
from jax.experimental import pallas as pl
from jax.experimental.pallas import tpu as pltpu
from jax.experimental.pallas import tpu_sc as plsc

N = 10000
NPAD = 10112
E = 320000
NC, NS = 2, 16
NW = NC * NS
CH = 128
K = 80
EPAD = NW * K * CH
RP = NPAD // NS
H = 64
DEGW = 16

_f32 = jnp.float32
_mesh = plsc.VectorSubcoreMesh(core_axis_name="c", subcore_axis_name="s")
_sc_params = pltpu.CompilerParams(use_tc_tiling_on_sc=False)


def _deg_body(dstm, z16, ones16, out, didx, ones_v, deg_sh):
    c = lax.axis_index("c")
    s = lax.axis_index("s")
    wid = c * NS + s
    pltpu.sync_copy(dstm.at[pl.ds(wid * K, K)], didx)
    pltpu.sync_copy(ones16, ones_v)
    pltpu.sync_copy(z16.at[pl.ds(s * RP, RP)], deg_sh.at[pl.ds(s * RP, RP)])
    plsc.subcore_barrier()

    def body(j, carry):
        pltpu.sync_copy(ones_v, deg_sh.at[didx.at[j]], add=True)
        return carry

    lax.fori_loop(0, K, body, 0)
    plsc.subcore_barrier()
    pltpu.sync_copy(deg_sh.at[pl.ds(s * RP, RP)],
                    out.at[pl.ds(c * NPAD + s * RP, RP)])


_deg_call = functools.partial(
    pl.kernel,
    out_type=jax.ShapeDtypeStruct((NC * NPAD, DEGW), _f32),
    mesh=_mesh,
    scratch_types=[
        pltpu.VMEM((K, CH), jnp.int32),
        pltpu.VMEM((CH, DEGW), _f32),
        pltpu.VMEM_SHARED((NPAD, DEGW), _f32),
    ],
    compiler_params=_sc_params,
)(_deg_body)


NBUF = 8


def _agg_body(h, srcm, dstm, z64, out, sidx, didx, rows, acc_sh, *sems):
    gsem = sems[:NBUF]
    ssem = sems[NBUF:]
    c = lax.axis_index("c")
    s = lax.axis_index("s")
    wid = c * NS + s
    pltpu.sync_copy(srcm.at[pl.ds(wid * K, K)], sidx)
    pltpu.sync_copy(dstm.at[pl.ds(wid * K, K)], didx)
    pltpu.sync_copy(z64.at[pl.ds(s * RP, RP)], acc_sh.at[pl.ds(s * RP, RP)])
    plsc.subcore_barrier()

    def rbuf(b):
        return rows.at[pl.ds(b * CH, CH)]

    for b in range(NBUF):
        pltpu.async_copy(h.at[sidx.at[b]], rbuf(b), gsem[b])

    def body(i, carry):
        base = i * NBUF
        for b in range(NBUF):
            j = base + b
            pltpu.make_async_copy(h.at[sidx.at[j]], rbuf(b), gsem[b]).wait()
            pltpu.async_copy(rbuf(b), acc_sh.at[didx.at[j]], ssem[b],
                             add=True)

        for b in range(NBUF):
            j = base + b

            @pl.when(j + NBUF < K)
            def _():
                pltpu.make_async_copy(
                    rbuf(b), acc_sh.at[didx.at[j]], ssem[b]).wait()
                pltpu.async_copy(h.at[sidx.at[j + NBUF]], rbuf(b), gsem[b])

        return carry

    lax.fori_loop(0, K // NBUF, body, 0)
    for b in range(NBUF):
        pltpu.make_async_copy(
            rbuf(b), acc_sh.at[didx.at[K - NBUF + b]], ssem[b]).wait()
    plsc.subcore_barrier()
    pltpu.sync_copy(acc_sh.at[pl.ds(s * RP, RP)],
                    out.at[pl.ds(c * NPAD + s * RP, RP)])


_agg_call = functools.partial(
    pl.kernel,
    out_type=jax.ShapeDtypeStruct((NC * NPAD, H), _f32),
    mesh=_mesh,
    scratch_types=[
        pltpu.VMEM((K, CH), jnp.int32),
        pltpu.VMEM((K, CH), jnp.int32),
        pltpu.VMEM((NBUF * CH, H), _f32),
        pltpu.VMEM_SHARED((NPAD, H), _f32),
    ] + [pltpu.SemaphoreType.DMA] * (2 * NBUF),
    compiler_params=_sc_params,
)(_agg_body)


def _tc0_body(x_ref, w_ref, b_ref, degp_ref, h_ref, invs_ref):
    deg = degp_ref[0:NPAD, 0:1] + degp_ref[NPAD:2 * NPAD, 0:1]
    invs = lax.rsqrt(jnp.maximum(deg, 1.0))
    hm = jnp.dot(x_ref[...], w_ref[...], preferred_element_type=_f32)
    h_ref[...] = (hm + b_ref[...]) * invs
    invs_ref[...] = invs


def _tc_layer_body(acc_ref, invs_ref, w_ref, b_ref, out_ref):
    invs = invs_ref[...]
    a = acc_ref[0:NPAD, :] + acc_ref[NPAD:2 * NPAD, :]
    hin = jnp.maximum(a * invs, 0.0)
    hm = jnp.dot(hin, w_ref[...], preferred_element_type=_f32)
    out_ref[...] = (hm + b_ref[...]) * invs


def _tc_final_body(acc_ref, invs_ref, batch_ref, wh_ref, bh_ref, wo_ref,
                   bo_ref, out_ref):
    a = acc_ref[0:NPAD, :] + acc_ref[NPAD:2 * NPAD, :]
    h5 = jnp.maximum(a * invs_ref[...], 0.0)
    gids = lax.broadcasted_iota(jnp.int32, (NPAD, 64), 1)
    onehot = (batch_ref[...] == gids).astype(_f32)
    psum = lax.dot_general(onehot, h5, (((0,), (0,)), ((), ())),
                           preferred_element_type=_f32)
    ones = jnp.ones((NPAD, 1), _f32)
    cnt = lax.dot_general(onehot, ones, (((0,), (0,)), ((), ())),
                          preferred_element_type=_f32)
    pooled = psum / jnp.maximum(cnt, 1.0)
    hid = jnp.maximum(
        jnp.dot(pooled, wh_ref[...], preferred_element_type=_f32)
        + bh_ref[...], 0.0)
    out_ref[...] = (jnp.dot(hid, wo_ref[...], preferred_element_type=_f32)
                    + bo_ref[...])


def _tc0(x_pad, w0, b0, degp):
    return pl.pallas_call(
        _tc0_body,
        out_shape=[
            jax.ShapeDtypeStruct((NPAD, H), _f32),
            jax.ShapeDtypeStruct((NPAD, 1), _f32),
        ],
    )(x_pad, w0, b0, degp)


def _tc_layer(acc, invs, w, b):
    return pl.pallas_call(
        _tc_layer_body,
        out_shape=jax.ShapeDtypeStruct((NPAD, H), _f32),
    )(acc, invs, w, b)


def _tc_final(acc, invs, batch_pad, wh, bh, wo, bo):
    return pl.pallas_call(
        _tc_final_body,
        out_shape=jax.ShapeDtypeStruct((64, 32), _f32),
    )(acc, invs, batch_pad, wh, bh, wo, bo)


def kernel(x, edge_index, batch, W0, b0, W1, b1, W2, b2, W3, b3, W4, b4,
           Wh, bh, Wo, bo):
    src = edge_index[0].astype(jnp.int32)
    dst = edge_index[1].astype(jnp.int32)
    pad = jnp.full((EPAD - E,), N, dtype=jnp.int32)
    srcm = jnp.concatenate([src, pad]).reshape(NW * K, CH)
    dstm = jnp.concatenate([dst, pad]).reshape(NW * K, CH)

    x_pad = jnp.concatenate(
        [x, jnp.zeros((NPAD - N, x.shape[1]), _f32)], axis=0)
    batch_pad = jnp.concatenate(
        [batch.astype(jnp.int32),
         jnp.full((NPAD - N,), 64, dtype=jnp.int32)]).reshape(NPAD, 1)

    z16 = jnp.zeros((NPAD, DEGW), _f32)
    z64 = jnp.zeros((NPAD, H), _f32)
    ones16 = jnp.ones((CH, DEGW), _f32)

    degp = _deg_call(dstm, z16, ones16)
    h, invs = _tc0(x_pad, W0, b0.reshape(1, H), degp)
    for (W, b) in ((W1, b1), (W2, b2), (W3, b3), (W4, b4)):
        acc = _agg_call(h, srcm, dstm, z64)
        h = _tc_layer(acc, invs, W, b.reshape(1, H))
    acc = _agg_call(h, srcm, dstm, z64)
    return _tc_final(acc, invs, batch_pad, Wh, bh.reshape(1, H),
                     Wo, bo.reshape(1, 32))


# speedup vs baseline: 2.3621x; 2.3621x over previous
import functools

import jax
import jax.numpy as jnp
from jax import lax
from jax.experimental import pallas as pl
from jax.experimental.pallas import tpu as pltpu
from jax.experimental.pallas import tpu_sc as plsc

N = 10000
NPAD = 10112
E = 320000
NC, NS = 2, 16
NW = NC * NS
CH = 128
K = 80
EPAD = NW * K * CH
RP = NPAD // NS
H = 64
DEGW = 16
NBUF = 4

_f32 = jnp.float32
_mesh = plsc.VectorSubcoreMesh(core_axis_name="c", subcore_axis_name="s")
_sc_params = pltpu.CompilerParams(use_tc_tiling_on_sc=False)


def _deg_body(dstm, z16, ones16, out, didx, ones_v, deg_sh, *sems):
    c = lax.axis_index("c")
    s = lax.axis_index("s")
    wid = c * NS + s
    pltpu.sync_copy(dstm.at[pl.ds(wid * K, K)], didx)
    pltpu.sync_copy(ones16, ones_v)
    pltpu.sync_copy(z16.at[pl.ds(s * RP, RP)], deg_sh.at[pl.ds(s * RP, RP)])
    plsc.subcore_barrier()

    def body(i, carry):
        base = i * NBUF
        for b in range(NBUF):
            j = base + b

            @pl.when(j >= NBUF)
            def _():
                pltpu.make_async_copy(
                    ones_v, deg_sh.at[didx.at[j - NBUF]], sems[b]).wait()

            pltpu.async_copy(ones_v, deg_sh.at[didx.at[j]], sems[b],
                             add=True)
        return carry

    lax.fori_loop(0, K // NBUF, body, 0)
    for b in range(NBUF):
        pltpu.make_async_copy(
            ones_v, deg_sh.at[didx.at[K - NBUF + b]], sems[b]).wait()
    plsc.subcore_barrier()
    pltpu.sync_copy(deg_sh.at[pl.ds(s * RP, RP)],
                    out.at[pl.ds(c * NPAD + s * RP, RP)])


_deg_call = functools.partial(
    pl.kernel,
    out_type=jax.ShapeDtypeStruct((NC * NPAD, DEGW), _f32),
    mesh=_mesh,
    scratch_types=[
        pltpu.VMEM((K, CH), jnp.int32),
        pltpu.VMEM((CH, DEGW), _f32),
        pltpu.VMEM_SHARED((NPAD, DEGW), _f32),
    ] + [pltpu.SemaphoreType.DMA] * NBUF,
    compiler_params=_sc_params,
)(_deg_body)


KP = K // 2


def _agg_body(h, srcm, dstm, z64, out, sidx, didx, rows, acc_sh, h_sh, *sems):
    gsem = sems[:NBUF]
    ssem = sems[NBUF:]
    c = lax.axis_index("c")
    s = lax.axis_index("s")
    wid = c * NS + s
    pltpu.sync_copy(z64.at[pl.ds(s * RP, RP)], acc_sh.at[pl.ds(s * RP, RP)])
    pltpu.sync_copy(h.at[pl.ds(s * RP, RP)], h_sh.at[pl.ds(s * RP, RP)])
    plsc.subcore_barrier()

    def rbuf(b):
        return rows.at[pl.ds(b * CH, CH)]

    for p in range(2):
        pltpu.sync_copy(srcm.at[pl.ds(wid * K + p * KP, KP)], sidx)
        pltpu.sync_copy(dstm.at[pl.ds(wid * K + p * KP, KP)], didx)

        for b in range(NBUF):
            pltpu.async_copy(h_sh.at[sidx.at[b]], rbuf(b), gsem[b])

        def body(i, carry):
            base = i * NBUF
            for b in range(NBUF):
                j = base + b
                pltpu.make_async_copy(h_sh.at[sidx.at[j]], rbuf(b),
                                      gsem[b]).wait()
                pltpu.async_copy(rbuf(b), acc_sh.at[didx.at[j]], ssem[b],
                                 add=True)

            for b in range(NBUF):
                j = base + b

                @pl.when(j + NBUF < KP)
                def _():
                    pltpu.make_async_copy(
                        rbuf(b), acc_sh.at[didx.at[j]], ssem[b]).wait()
                    pltpu.async_copy(h_sh.at[sidx.at[j + NBUF]], rbuf(b),
                                     gsem[b])

            return carry

        lax.fori_loop(0, KP // NBUF, body, 0)
        for b in range(NBUF):
            pltpu.make_async_copy(
                rbuf(b), acc_sh.at[didx.at[KP - NBUF + b]], ssem[b]).wait()

    plsc.subcore_barrier()
    pltpu.sync_copy(acc_sh.at[pl.ds(s * RP, RP)],
                    out.at[pl.ds(c * NPAD + s * RP, RP)])


_agg_call = functools.partial(
    pl.kernel,
    out_type=jax.ShapeDtypeStruct((NC * NPAD, H), _f32),
    mesh=_mesh,
    scratch_types=[
        pltpu.VMEM((K // 2, CH), jnp.int32),
        pltpu.VMEM((K // 2, CH), jnp.int32),
        pltpu.VMEM((NBUF * CH, H), _f32),
        pltpu.VMEM_SHARED((NPAD, H), _f32),
        pltpu.VMEM_SHARED((NPAD, H), _f32),
    ] + [pltpu.SemaphoreType.DMA] * (2 * NBUF),
    compiler_params=_sc_params,
)(_agg_body)


def _tc0_body(x_ref, w_ref, b_ref, degp_ref, h_ref, invs_ref):
    deg = degp_ref[0:NPAD, 0:1] + degp_ref[NPAD:2 * NPAD, 0:1]
    invs = lax.rsqrt(jnp.maximum(deg, 1.0))
    hm = jnp.dot(x_ref[...], w_ref[...], preferred_element_type=_f32)
    hm = jnp.concatenate([hm, jnp.zeros((NPAD - N, H), _f32)], axis=0)
    h_ref[...] = (hm + b_ref[...]) * invs
    invs_ref[...] = invs


def _tc_layer_body(acc_ref, invs_ref, w_ref, b_ref, out_ref):
    invs = invs_ref[...]
    a = acc_ref[0:NPAD, :] + acc_ref[NPAD:2 * NPAD, :]
    hin = jnp.maximum(a * invs, 0.0)
    hm = jnp.dot(hin, w_ref[...], preferred_element_type=_f32)
    out_ref[...] = (hm + b_ref[...]) * invs


def _tc_final_body(acc_ref, invs_ref, batch_ref, wh_ref, bh_ref, wo_ref,
                   bo_ref, out_ref):
    a = acc_ref[0:NPAD, :] + acc_ref[NPAD:2 * NPAD, :]
    h5 = jnp.maximum(a * invs_ref[...], 0.0)
    gids = lax.broadcasted_iota(jnp.int32, (NPAD, 64), 1)
    onehot = (batch_ref[...] == gids).astype(_f32)
    psum = lax.dot_general(onehot, h5, (((0,), (0,)), ((), ())),
                           preferred_element_type=_f32)
    ones = jnp.ones((NPAD, 1), _f32)
    cnt = lax.dot_general(onehot, ones, (((0,), (0,)), ((), ())),
                          preferred_element_type=_f32)
    pooled = psum / jnp.maximum(cnt, 1.0)
    hid = jnp.maximum(
        jnp.dot(pooled, wh_ref[...], preferred_element_type=_f32)
        + bh_ref[...], 0.0)
    out_ref[...] = (jnp.dot(hid, wo_ref[...], preferred_element_type=_f32)
                    + bo_ref[...])


def _tc0(x, w0, b0, degp):
    return pl.pallas_call(
        _tc0_body,
        out_shape=[
            jax.ShapeDtypeStruct((NPAD, H), _f32),
            jax.ShapeDtypeStruct((NPAD, 1), _f32),
        ],
    )(x, w0, b0, degp)


def _tc_layer(acc, invs, w, b):
    return pl.pallas_call(
        _tc_layer_body,
        out_shape=jax.ShapeDtypeStruct((NPAD, H), _f32),
    )(acc, invs, w, b)


def _tc_final(acc, invs, batch_pad, wh, bh, wo, bo):
    return pl.pallas_call(
        _tc_final_body,
        out_shape=jax.ShapeDtypeStruct((64, 32), _f32),
    )(acc, invs, batch_pad, wh, bh, wo, bo)


def kernel(x, edge_index, batch, W0, b0, W1, b1, W2, b2, W3, b3, W4, b4,
           Wh, bh, Wo, bo):
    src = edge_index[0].astype(jnp.int32)
    dst = edge_index[1].astype(jnp.int32)
    pad = jnp.full((EPAD - E,), N, dtype=jnp.int32)
    srcm = jnp.concatenate([src, pad]).reshape(NW * K, CH)
    dstm = jnp.concatenate([dst, pad]).reshape(NW * K, CH)

    batch_pad = jnp.concatenate(
        [batch.astype(jnp.int32),
         jnp.full((NPAD - N,), 64, dtype=jnp.int32)]).reshape(NPAD, 1)

    z16 = jnp.zeros((NPAD, DEGW), _f32)
    z64 = jnp.zeros((NPAD, H), _f32)
    ones16 = jnp.ones((CH, DEGW), _f32)

    degp = _deg_call(dstm, z16, ones16)
    h, invs = _tc0(x, W0, b0.reshape(1, H), degp)
    for (W, b) in ((W1, b1), (W2, b2), (W3, b3), (W4, b4)):
        acc = _agg_call(h, srcm, dstm, z64)
        h = _tc_layer(acc, invs, W, b.reshape(1, H))
    acc = _agg_call(h, srcm, dstm, z64)
    return _tc_final(acc, invs, batch_pad, Wh, bh.reshape(1, H),
                     Wo, bo.reshape(1, 32))


# speedup vs baseline: 2.4812x; 1.0504x over previous
import functools

import jax
import jax.numpy as jnp
from jax import lax
from jax.experimental import pallas as pl
from jax.experimental.pallas import tpu as pltpu
from jax.experimental.pallas import tpu_sc as plsc

N = 10000
NPAD = 10112
E = 320000
NC, NS = 2, 16
NW = NC * NS
CH = 128
K = 80
EPAD = NW * K * CH
RP = NPAD // NS
H = 64
DEGW = 16
NBUF = 2

_f32 = jnp.float32
_mesh = plsc.VectorSubcoreMesh(core_axis_name="c", subcore_axis_name="s")
_sc_params = pltpu.CompilerParams(use_tc_tiling_on_sc=False,
                                  skip_device_barrier=True)


def _deg_body(dstm, z16, ones16, out, didx, ones_v, deg_sh, *sems):
    c = lax.axis_index("c")
    s = lax.axis_index("s")
    wid = c * NS + s
    pltpu.sync_copy(dstm.at[pl.ds(wid * K, K)], didx)
    pltpu.sync_copy(ones16, ones_v)
    pltpu.sync_copy(z16.at[pl.ds(s * RP, RP)], deg_sh.at[pl.ds(s * RP, RP)])
    plsc.subcore_barrier()

    def body(i, carry):
        base = i * NBUF
        for b in range(NBUF):
            j = base + b

            @pl.when(j >= NBUF)
            def _():
                pltpu.make_async_copy(
                    ones_v, deg_sh.at[didx.at[j - NBUF]], sems[b]).wait()

            pltpu.async_copy(ones_v, deg_sh.at[didx.at[j]], sems[b],
                             add=True)
        return carry

    lax.fori_loop(0, K // NBUF, body, 0)
    for b in range(NBUF):
        pltpu.make_async_copy(
            ones_v, deg_sh.at[didx.at[K - NBUF + b]], sems[b]).wait()
    plsc.subcore_barrier()
    pltpu.sync_copy(deg_sh.at[pl.ds(s * RP, RP)],
                    out.at[pl.ds(c * NPAD + s * RP, RP)])


_deg_call = functools.partial(
    pl.kernel,
    out_type=jax.ShapeDtypeStruct((NC * NPAD, DEGW), _f32),
    mesh=_mesh,
    scratch_types=[
        pltpu.VMEM((K, CH), jnp.int32),
        pltpu.VMEM((CH, DEGW), _f32),
        pltpu.VMEM_SHARED((NPAD, DEGW), _f32),
    ] + [pltpu.SemaphoreType.DMA] * NBUF,
    compiler_params=_sc_params,
)(_deg_body)


def _agg_body(h, srcm, dstm, z64, out, sidx, didx, rows, acc_sh, h_sh, *sems):
    gsem = sems[:NBUF]
    ssem = sems[NBUF:]
    c = lax.axis_index("c")
    s = lax.axis_index("s")
    wid = c * NS + s
    pltpu.sync_copy(srcm.at[pl.ds(wid * K, K)], sidx)
    pltpu.sync_copy(dstm.at[pl.ds(wid * K, K)], didx)
    pltpu.sync_copy(z64.at[pl.ds(s * RP, RP)], acc_sh.at[pl.ds(s * RP, RP)])
    pltpu.sync_copy(h.at[pl.ds(s * RP, RP)], h_sh.at[pl.ds(s * RP, RP)])
    plsc.subcore_barrier()

    def rbuf(b):
        return rows.at[pl.ds(b * CH, CH)]

    for b in range(NBUF):
        pltpu.async_copy(h_sh.at[sidx.at[b]], rbuf(b), gsem[b])

    def body(i, carry):
        base = i * NBUF
        for b in range(NBUF):
            j = base + b
            pltpu.make_async_copy(h_sh.at[sidx.at[j]], rbuf(b),
                                  gsem[b]).wait()
            pltpu.async_copy(rbuf(b), acc_sh.at[didx.at[j]], ssem[b],
                             add=True)

        for b in range(NBUF):
            j = base + b

            @pl.when(j + NBUF < K)
            def _():
                pltpu.make_async_copy(
                    rbuf(b), acc_sh.at[didx.at[j]], ssem[b]).wait()
                pltpu.async_copy(h_sh.at[sidx.at[j + NBUF]], rbuf(b),
                                 gsem[b])

        return carry

    lax.fori_loop(0, K // NBUF, body, 0)
    for b in range(NBUF):
        pltpu.make_async_copy(
            rbuf(b), acc_sh.at[didx.at[K - NBUF + b]], ssem[b]).wait()
    plsc.subcore_barrier()
    pltpu.sync_copy(acc_sh.at[pl.ds(s * RP, RP)],
                    out.at[pl.ds(c * NPAD + s * RP, RP)])


_agg_call = functools.partial(
    pl.kernel,
    out_type=jax.ShapeDtypeStruct((NC * NPAD, H), _f32),
    mesh=_mesh,
    scratch_types=[
        pltpu.VMEM((K, CH), jnp.int32),
        pltpu.VMEM((K, CH), jnp.int32),
        pltpu.VMEM((NBUF * CH, H), _f32),
        pltpu.VMEM_SHARED((NPAD, H), _f32),
        pltpu.VMEM_SHARED((NPAD, H), _f32),
    ] + [pltpu.SemaphoreType.DMA] * (2 * NBUF),
    compiler_params=_sc_params,
)(_agg_body)


def _tc0_body(x_ref, w_ref, b_ref, degp_ref, h_ref, invs_ref):
    deg = degp_ref[0:NPAD, 0:1] + degp_ref[NPAD:2 * NPAD, 0:1]
    invs = lax.rsqrt(jnp.maximum(deg, 1.0))
    hm = jnp.dot(x_ref[...], w_ref[...], preferred_element_type=_f32)
    hm = jnp.concatenate([hm, jnp.zeros((NPAD - N, H), _f32)], axis=0)
    h_ref[...] = (hm + b_ref[...]) * invs
    invs_ref[...] = invs


def _tc_layer_body(acc_ref, invs_ref, w_ref, b_ref, out_ref):
    invs = invs_ref[...]
    a = acc_ref[0:NPAD, :] + acc_ref[NPAD:2 * NPAD, :]
    hin = jnp.maximum(a * invs, 0.0)
    hm = jnp.dot(hin, w_ref[...], preferred_element_type=_f32)
    out_ref[...] = (hm + b_ref[...]) * invs


def _tc_final_body(acc_ref, invs_ref, batch_ref, wh_ref, bh_ref, wo_ref,
                   bo_ref, out_ref):
    a = acc_ref[0:NPAD, :] + acc_ref[NPAD:2 * NPAD, :]
    h5 = jnp.maximum(a * invs_ref[...], 0.0)
    gids = lax.broadcasted_iota(jnp.int32, (NPAD, 64), 1)
    onehot = (batch_ref[...] == gids).astype(_f32)
    psum = lax.dot_general(onehot, h5, (((0,), (0,)), ((), ())),
                           preferred_element_type=_f32)
    ones = jnp.ones((NPAD, 1), _f32)
    cnt = lax.dot_general(onehot, ones, (((0,), (0,)), ((), ())),
                          preferred_element_type=_f32)
    pooled = psum / jnp.maximum(cnt, 1.0)
    hid = jnp.maximum(
        jnp.dot(pooled, wh_ref[...], preferred_element_type=_f32)
        + bh_ref[...], 0.0)
    out_ref[...] = (jnp.dot(hid, wo_ref[...], preferred_element_type=_f32)
                    + bo_ref[...])


def _tc0(x, w0, b0, degp):
    return pl.pallas_call(
        _tc0_body,
        out_shape=[
            jax.ShapeDtypeStruct((NPAD, H), _f32),
            jax.ShapeDtypeStruct((NPAD, 1), _f32),
        ],
    )(x, w0, b0, degp)


def _tc_layer(acc, invs, w, b):
    return pl.pallas_call(
        _tc_layer_body,
        out_shape=jax.ShapeDtypeStruct((NPAD, H), _f32),
    )(acc, invs, w, b)


def _tc_final(acc, invs, batch_pad, wh, bh, wo, bo):
    return pl.pallas_call(
        _tc_final_body,
        out_shape=jax.ShapeDtypeStruct((64, 32), _f32),
    )(acc, invs, batch_pad, wh, bh, wo, bo)


def kernel(x, edge_index, batch, W0, b0, W1, b1, W2, b2, W3, b3, W4, b4,
           Wh, bh, Wo, bo):
    src = edge_index[0].astype(jnp.int32)
    dst = edge_index[1].astype(jnp.int32)
    pad = jnp.full((EPAD - E,), N, dtype=jnp.int32)
    srcm = jnp.concatenate([src, pad]).reshape(NW * K, CH)
    dstm = jnp.concatenate([dst, pad]).reshape(NW * K, CH)

    batch_pad = jnp.concatenate(
        [batch.astype(jnp.int32),
         jnp.full((NPAD - N,), 64, dtype=jnp.int32)]).reshape(NPAD, 1)

    z16 = jnp.zeros((NPAD, DEGW), _f32)
    z64 = jnp.zeros((NPAD, H), _f32)
    ones16 = jnp.ones((CH, DEGW), _f32)

    degp = _deg_call(dstm, z16, ones16)
    h, invs = _tc0(x, W0, b0.reshape(1, H), degp)
    for (W, b) in ((W1, b1), (W2, b2), (W3, b3), (W4, b4)):
        acc = _agg_call(h, srcm, dstm, z64)
        h = _tc_layer(acc, invs, W, b.reshape(1, H))
    acc = _agg_call(h, srcm, dstm, z64)
    return _tc_final(acc, invs, batch_pad, Wh, bh.reshape(1, H),
                     Wo, bo.reshape(1, 32))


# speedup vs baseline: 2.4967x; 1.0062x over previous
import functools

import jax
import jax.numpy as jnp
from jax import lax
from jax.experimental import pallas as pl
from jax.experimental.pallas import tpu as pltpu
from jax.experimental.pallas import tpu_sc as plsc

N = 10000
NPAD = 10112
E = 320000
NC, NS = 2, 16
NW = NC * NS
CH = 128
K = 80
EPAD = NW * K * CH
RP = NPAD // NS
H = 64
DEGW = 16
NBUF = 2

_f32 = jnp.float32
_mesh = plsc.VectorSubcoreMesh(core_axis_name="c", subcore_axis_name="s")
_sc_params = pltpu.CompilerParams(use_tc_tiling_on_sc=False,
                                  skip_device_barrier=True)


def _deg_body(dstm, z16, ones16, out, didx, ones_v, deg_sh, *sems):
    c = lax.axis_index("c")
    s = lax.axis_index("s")
    wid = c * NS + s
    pltpu.sync_copy(dstm.at[pl.ds(wid * K, K)], didx)
    pltpu.sync_copy(ones16, ones_v)
    pltpu.sync_copy(z16.at[pl.ds(s * RP, RP)], deg_sh.at[pl.ds(s * RP, RP)])
    plsc.subcore_barrier()

    def body(i, carry):
        base = i * NBUF
        for b in range(NBUF):
            j = base + b

            @pl.when(j >= NBUF)
            def _():
                pltpu.make_async_copy(
                    ones_v, deg_sh.at[didx.at[j - NBUF]], sems[b]).wait()

            pltpu.async_copy(ones_v, deg_sh.at[didx.at[j]], sems[b],
                             add=True)
        return carry

    lax.fori_loop(0, K // NBUF, body, 0)
    for b in range(NBUF):
        pltpu.make_async_copy(
            ones_v, deg_sh.at[didx.at[K - NBUF + b]], sems[b]).wait()
    plsc.subcore_barrier()
    pltpu.sync_copy(deg_sh.at[pl.ds(s * RP, RP)],
                    out.at[pl.ds(c * NPAD + s * RP, RP)])


_deg_call = functools.partial(
    pl.kernel,
    out_type=jax.ShapeDtypeStruct((NC * NPAD, DEGW), _f32),
    mesh=_mesh,
    scratch_types=[
        pltpu.VMEM((K, CH), jnp.int32),
        pltpu.VMEM((CH, DEGW), _f32),
        pltpu.VMEM_SHARED((NPAD, DEGW), _f32),
    ] + [pltpu.SemaphoreType.DMA] * NBUF,
    compiler_params=_sc_params,
)(_deg_body)


def _agg_body(h, srcm, dstm, z64, out, sidx, didx, rows, acc_sh, h_sh, *sems):
    gsem = sems[:NBUF]
    ssem = sems[NBUF:]
    c = lax.axis_index("c")
    s = lax.axis_index("s")
    wid = c * NS + s
    cp0 = pltpu.async_copy(srcm.at[pl.ds(wid * K, K)], sidx, gsem[0])
    cp1 = pltpu.async_copy(dstm.at[pl.ds(wid * K, K)], didx, gsem[1])
    cp2 = pltpu.async_copy(z64.at[pl.ds(s * RP, RP)],
                           acc_sh.at[pl.ds(s * RP, RP)], ssem[0])
    cp3 = pltpu.async_copy(h.at[pl.ds(s * RP, RP)],
                           h_sh.at[pl.ds(s * RP, RP)], ssem[1])
    cp0.wait()
    cp1.wait()
    cp2.wait()
    cp3.wait()
    plsc.subcore_barrier()

    def rbuf(b):
        return rows.at[pl.ds(b * CH, CH)]

    for b in range(NBUF):
        pltpu.async_copy(h_sh.at[sidx.at[b]], rbuf(b), gsem[b])

    def body(i, carry):
        base = i * NBUF
        for b in range(NBUF):
            j = base + b
            pltpu.make_async_copy(h_sh.at[sidx.at[j]], rbuf(b),
                                  gsem[b]).wait()
            pltpu.async_copy(rbuf(b), acc_sh.at[didx.at[j]], ssem[b],
                             add=True)

        for b in range(NBUF):
            j = base + b

            @pl.when(j + NBUF < K)
            def _():
                pltpu.make_async_copy(
                    rbuf(b), acc_sh.at[didx.at[j]], ssem[b]).wait()
                pltpu.async_copy(h_sh.at[sidx.at[j + NBUF]], rbuf(b),
                                 gsem[b])

        return carry

    lax.fori_loop(0, K // NBUF, body, 0)
    for b in range(NBUF):
        pltpu.make_async_copy(
            rbuf(b), acc_sh.at[didx.at[K - NBUF + b]], ssem[b]).wait()
    plsc.subcore_barrier()
    pltpu.sync_copy(acc_sh.at[pl.ds(s * RP, RP)],
                    out.at[pl.ds(c * NPAD + s * RP, RP)])


_agg_call = functools.partial(
    pl.kernel,
    out_type=jax.ShapeDtypeStruct((NC * NPAD, H), _f32),
    mesh=_mesh,
    scratch_types=[
        pltpu.VMEM((K, CH), jnp.int32),
        pltpu.VMEM((K, CH), jnp.int32),
        pltpu.VMEM((NBUF * CH, H), _f32),
        pltpu.VMEM_SHARED((NPAD, H), _f32),
        pltpu.VMEM_SHARED((NPAD, H), _f32),
    ] + [pltpu.SemaphoreType.DMA] * (2 * NBUF),
    compiler_params=_sc_params,
)(_agg_body)


def _tc0_body(x_ref, w_ref, b_ref, degp_ref, h_ref, invs_ref):
    deg = degp_ref[0:NPAD, 0:1] + degp_ref[NPAD:2 * NPAD, 0:1]
    invs = lax.rsqrt(jnp.maximum(deg, 1.0))
    hm = jnp.dot(x_ref[...], w_ref[...], preferred_element_type=_f32)
    hm = jnp.concatenate([hm, jnp.zeros((NPAD - N, H), _f32)], axis=0)
    h_ref[...] = (hm + b_ref[...]) * invs
    invs_ref[...] = invs


def _tc_layer_body(acc_ref, invs_ref, w_ref, b_ref, out_ref):
    invs = invs_ref[...]
    a = acc_ref[0:NPAD, :] + acc_ref[NPAD:2 * NPAD, :]
    hin = jnp.maximum(a * invs, 0.0)
    hm = jnp.dot(hin, w_ref[...], preferred_element_type=_f32)
    out_ref[...] = (hm + b_ref[...]) * invs


def _tc_final_body(acc_ref, invs_ref, batch_ref, wh_ref, bh_ref, wo_ref,
                   bo_ref, out_ref):
    a = acc_ref[0:NPAD, :] + acc_ref[NPAD:2 * NPAD, :]
    h5 = jnp.maximum(a * invs_ref[...], 0.0)
    gids = lax.broadcasted_iota(jnp.int32, (NPAD, 64), 1)
    onehot = (batch_ref[...] == gids).astype(_f32)
    psum = lax.dot_general(onehot, h5, (((0,), (0,)), ((), ())),
                           preferred_element_type=_f32)
    ones = jnp.ones((NPAD, 1), _f32)
    cnt = lax.dot_general(onehot, ones, (((0,), (0,)), ((), ())),
                          preferred_element_type=_f32)
    pooled = psum / jnp.maximum(cnt, 1.0)
    hid = jnp.maximum(
        jnp.dot(pooled, wh_ref[...], preferred_element_type=_f32)
        + bh_ref[...], 0.0)
    out_ref[...] = (jnp.dot(hid, wo_ref[...], preferred_element_type=_f32)
                    + bo_ref[...])


def _tc0(x, w0, b0, degp):
    return pl.pallas_call(
        _tc0_body,
        out_shape=[
            jax.ShapeDtypeStruct((NPAD, H), _f32),
            jax.ShapeDtypeStruct((NPAD, 1), _f32),
        ],
    )(x, w0, b0, degp)


def _tc_layer(acc, invs, w, b):
    return pl.pallas_call(
        _tc_layer_body,
        out_shape=jax.ShapeDtypeStruct((NPAD, H), _f32),
    )(acc, invs, w, b)


def _tc_final(acc, invs, batch_pad, wh, bh, wo, bo):
    return pl.pallas_call(
        _tc_final_body,
        out_shape=jax.ShapeDtypeStruct((64, 32), _f32),
    )(acc, invs, batch_pad, wh, bh, wo, bo)


def kernel(x, edge_index, batch, W0, b0, W1, b1, W2, b2, W3, b3, W4, b4,
           Wh, bh, Wo, bo):
    src = edge_index[0].astype(jnp.int32)
    dst = edge_index[1].astype(jnp.int32)
    pad = jnp.full((EPAD - E,), N, dtype=jnp.int32)
    srcm = jnp.concatenate([src, pad]).reshape(NW * K, CH)
    dstm = jnp.concatenate([dst, pad]).reshape(NW * K, CH)

    batch_pad = jnp.concatenate(
        [batch.astype(jnp.int32),
         jnp.full((NPAD - N,), 64, dtype=jnp.int32)]).reshape(NPAD, 1)

    z16 = jnp.zeros((NPAD, DEGW), _f32)
    z64 = jnp.zeros((NPAD, H), _f32)
    ones16 = jnp.ones((CH, DEGW), _f32)

    degp = _deg_call(dstm, z16, ones16)
    h, invs = _tc0(x, W0, b0.reshape(1, H), degp)
    for (W, b) in ((W1, b1), (W2, b2), (W3, b3), (W4, b4)):
        acc = _agg_call(h, srcm, dstm, z64)
        h = _tc_layer(acc, invs, W, b.reshape(1, H))
    acc = _agg_call(h, srcm, dstm, z64)
    return _tc_final(acc, invs, batch_pad, Wh, bh.reshape(1, H),
                     Wo, bo.reshape(1, 32))
